# baseline (speedup 1.0000x reference)
import functools

import jax
import jax.numpy as jnp
from jax.experimental import pallas as pl

N = 4096
H = 256
EPS = 0.3
BM = 512


def _embed_body(f_ref, w_ref, b_ref, out_ref):
    acc = jnp.dot(f_ref[...], w_ref[...], preferred_element_type=jnp.float32)
    out_ref[...] = jnp.maximum(acc + b_ref[...], 0.0)


def _embed(feature, lin_w, lin_b):
    n, f_in = feature.shape
    h = lin_w.shape[1]
    return pl.pallas_call(
        _embed_body,
        grid=(n // BM,),
        in_specs=[
            pl.BlockSpec((BM, f_in), lambda i: (i, 0)),
            pl.BlockSpec((f_in, h), lambda i: (0, 0)),
            pl.BlockSpec((1, h), lambda i: (0, 0)),
        ],
        out_specs=pl.BlockSpec((BM, h), lambda i: (i, 0)),
        out_shape=jax.ShapeDtypeStruct((n, h), jnp.float32),
    )(feature, lin_w, lin_b.reshape(1, h))


def _fa_body(h_ref, adj_ref, wgd_ref, wgs_ref, bg_ref, h0_ref, out_ref):
    i = pl.program_id(0)
    h = h_ref[...]
    hi = h_ref[pl.ds(i * BM, BM), :]
    a = jnp.dot(hi, wgd_ref[...], preferred_element_type=jnp.float32)
    b = jnp.dot(h, wgs_ref[...], preferred_element_type=jnp.float32)
    g = jnp.tanh(a + b.reshape(1, N) + bg_ref[0, 0])
    e = adj_ref[...] * g
    acc = jnp.dot(e, h, preferred_element_type=jnp.float32)
    out_ref[...] = jnp.maximum(acc, 0.0) + EPS * h0_ref[...]


def _fa_layer(h, adj, wg_dst, wg_src, bg, h0):
    return pl.pallas_call(
        _fa_body,
        grid=(N // BM,),
        in_specs=[
            pl.BlockSpec((N, H), lambda i: (0, 0)),
            pl.BlockSpec((BM, N), lambda i: (i, 0)),
            pl.BlockSpec((H, 1), lambda i: (0, 0)),
            pl.BlockSpec((H, 1), lambda i: (0, 0)),
            pl.BlockSpec((1, 1), lambda i: (0, 0)),
            pl.BlockSpec((BM, H), lambda i: (i, 0)),
        ],
        out_specs=pl.BlockSpec((BM, H), lambda i: (i, 0)),
        out_shape=jax.ShapeDtypeStruct((N, H), jnp.float32),
    )(h, adj, wg_dst, wg_src, bg, h0)


def _fc_body(x_ref, w_ref, b_ref, out_ref):
    o = jnp.dot(x_ref[...], w_ref[...], preferred_element_type=jnp.float32)
    o = o + b_ref[...]
    m = jnp.max(o, axis=1, keepdims=True)
    lse = jnp.log(jnp.sum(jnp.exp(o - m), axis=1, keepdims=True))
    out_ref[...] = o - m - lse


def _fc(x, fc_w, fc_b):
    h, c = fc_w.shape
    return pl.pallas_call(
        _fc_body,
        grid=(N // BM,),
        in_specs=[
            pl.BlockSpec((BM, h), lambda i: (i, 0)),
            pl.BlockSpec((h, c), lambda i: (0, 0)),
            pl.BlockSpec((1, c), lambda i: (0, 0)),
        ],
        out_specs=pl.BlockSpec((BM, c), lambda i: (i, 0)),
        out_shape=jax.ShapeDtypeStruct((N, c), jnp.float32),
    )(x, fc_w, fc_b.reshape(1, c))


@jax.jit
def kernel(feature, adj, lin_w, lin_b, gate_w, gate_b, fc_w, fc_b):
    x = _embed(feature, lin_w, lin_b)
    h0 = x
    n_layer = gate_w.shape[0]
    hh = gate_w.shape[1] // 2
    for i in range(n_layer):
        wg_dst = gate_w[i, :hh].reshape(hh, 1)
        wg_src = gate_w[i, hh:].reshape(hh, 1)
        bg = gate_b[i].reshape(1, 1)
        x = _fa_layer(x, adj, wg_dst, wg_src, bg, h0)
    return _fc(x, fc_w, fc_b)
